# baseline (speedup 1.0000x reference)
import jax
import jax.numpy as jnp
from jax.experimental import pallas as pl


def _copy_body(w_ref, o_ref):
    o_ref[...] = w_ref[...]


def kernel(indices, weight):
    n = indices.shape[-2]
    d = weight.shape[-1]
    blk = 512
    return pl.pallas_call(
        _copy_body,
        grid=(n // blk,),
        out_shape=jax.ShapeDtypeStruct((n, d), weight.dtype),
        in_specs=[pl.BlockSpec((blk, d), lambda i: (i, 0))],
        out_specs=pl.BlockSpec((blk, d), lambda i: (i, 0)),
    )(weight)
